# baseline (device time: 63647 ns/iter reference)
import jax
import jax.numpy as jnp
from jax import lax
from jax.experimental import pallas as pl
from jax.experimental.pallas import tpu as pltpu

N_DEV = 16
M = 768
N = 768
CHUNK = M // N_DEV


def kernel(A, B):
    def body(a_ref, b_ref, out_ref, acc_ref, rs_buf, send_sems, recv_sems,
             credit_sem):
        my = lax.axis_index("i")

        barrier_sem = pltpu.get_barrier_semaphore()
        for off in range(1, N_DEV):
            pl.semaphore_signal(
                barrier_sem, inc=1,
                device_id=((my + off) % N_DEV,),
                device_id_type=pl.DeviceIdType.MESH,
            )
        pl.semaphore_wait(barrier_sem, N_DEV - 1)

        acc_ref[...] = jnp.dot(
            a_ref[...], b_ref[...], preferred_element_type=jnp.float32
        )

        phase1 = []
        for off in range(1, N_DEV):
            p = (my + off) % N_DEV
            rdma = pltpu.make_async_remote_copy(
                src_ref=acc_ref.at[pl.ds(p * CHUNK, CHUNK), :],
                dst_ref=rs_buf.at[my],
                send_sem=send_sems.at[off - 1],
                recv_sem=recv_sems.at[my],
                device_id=(p,),
                device_id_type=pl.DeviceIdType.MESH,
            )
            rdma.start()
            phase1.append(rdma)

        total = acc_ref[pl.ds(my * CHUNK, CHUNK), :]
        for off in range(1, N_DEV):
            q = (my - off) % N_DEV
            recv = pltpu.make_async_remote_copy(
                src_ref=rs_buf.at[q],
                dst_ref=rs_buf.at[q],
                send_sem=send_sems.at[0],
                recv_sem=recv_sems.at[q],
                device_id=(q,),
                device_id_type=pl.DeviceIdType.MESH,
            )
            recv.wait_recv()
            total = total + rs_buf[q]
        out_ref[pl.ds(my * CHUNK, CHUNK), :] = total

        for rdma in phase1:
            rdma.wait_send()

        for off in range(1, N_DEV):
            pl.semaphore_signal(
                credit_sem, inc=1,
                device_id=((my + off) % N_DEV,),
                device_id_type=pl.DeviceIdType.MESH,
            )
        pl.semaphore_wait(credit_sem, N_DEV - 1)

        phase2 = []
        for off in range(1, N_DEV):
            p = (my + off) % N_DEV
            rdma = pltpu.make_async_remote_copy(
                src_ref=out_ref.at[pl.ds(my * CHUNK, CHUNK), :],
                dst_ref=out_ref.at[pl.ds(my * CHUNK, CHUNK), :],
                send_sem=send_sems.at[off - 1],
                recv_sem=recv_sems.at[my],
                device_id=(p,),
                device_id_type=pl.DeviceIdType.MESH,
            )
            rdma.start()
            phase2.append(rdma)

        for off in range(1, N_DEV):
            q = (my - off) % N_DEV
            recv = pltpu.make_async_remote_copy(
                src_ref=out_ref.at[pl.ds(q * CHUNK, CHUNK), :],
                dst_ref=out_ref.at[pl.ds(q * CHUNK, CHUNK), :],
                send_sem=send_sems.at[0],
                recv_sem=recv_sems.at[q],
                device_id=(q,),
                device_id_type=pl.DeviceIdType.MESH,
            )
            recv.wait_recv()
        for rdma in phase2:
            rdma.wait_send()

    return pl.pallas_call(
        body,
        out_shape=jax.ShapeDtypeStruct((M, N), jnp.float32),
        in_specs=[
            pl.BlockSpec(memory_space=pltpu.VMEM),
            pl.BlockSpec(memory_space=pltpu.VMEM),
        ],
        out_specs=pl.BlockSpec(memory_space=pltpu.VMEM),
        scratch_shapes=[
            pltpu.VMEM((M, N), jnp.float32),
            pltpu.VMEM((N_DEV, CHUNK, N), jnp.float32),
            pltpu.SemaphoreType.DMA((N_DEV,)),
            pltpu.SemaphoreType.DMA((N_DEV,)),
            pltpu.SemaphoreType.REGULAR,
        ],
        compiler_params=pltpu.CompilerParams(collective_id=0),
    )(A, B)


# device time: 59912 ns/iter; 1.0623x vs baseline; 1.0623x over previous
import jax
import jax.numpy as jnp
from jax import lax
from jax.experimental import pallas as pl
from jax.experimental.pallas import tpu as pltpu

N_DEV = 16
M = 768
N = 768
CHUNK = M // N_DEV


def kernel(A, B):
    def body(a_ref, b_ref, out_ref, acc_ref, rs_buf, send_sems, recv_sems,
             send_sems2, recv_sems2):
        my = lax.axis_index("i")

        barrier_sem = pltpu.get_barrier_semaphore()
        for off in range(1, N_DEV):
            pl.semaphore_signal(
                barrier_sem, inc=1,
                device_id=((my + off) % N_DEV,),
                device_id_type=pl.DeviceIdType.MESH,
            )
        pl.semaphore_wait(barrier_sem, N_DEV - 1)

        acc_ref[...] = jnp.dot(
            a_ref[...], b_ref[...], preferred_element_type=jnp.float32
        )

        phase1 = []
        for off in range(1, N_DEV):
            p = (my + off) % N_DEV
            rdma = pltpu.make_async_remote_copy(
                src_ref=acc_ref.at[pl.ds(p * CHUNK, CHUNK), :],
                dst_ref=rs_buf.at[my],
                send_sem=send_sems.at[off - 1],
                recv_sem=recv_sems.at[my],
                device_id=(p,),
                device_id_type=pl.DeviceIdType.MESH,
            )
            rdma.start()
            phase1.append(rdma)

        total = acc_ref[pl.ds(my * CHUNK, CHUNK), :]
        for off in range(1, N_DEV):
            q = (my - off) % N_DEV
            recv = pltpu.make_async_remote_copy(
                src_ref=rs_buf.at[q],
                dst_ref=rs_buf.at[q],
                send_sem=send_sems.at[0],
                recv_sem=recv_sems.at[q],
                device_id=(q,),
                device_id_type=pl.DeviceIdType.MESH,
            )
            recv.wait_recv()
            total = total + rs_buf[q]
        out_ref[pl.ds(my * CHUNK, CHUNK), :] = total

        phase2 = []
        for off in range(1, N_DEV):
            p = (my + off) % N_DEV
            rdma = pltpu.make_async_remote_copy(
                src_ref=out_ref.at[pl.ds(my * CHUNK, CHUNK), :],
                dst_ref=out_ref.at[pl.ds(my * CHUNK, CHUNK), :],
                send_sem=send_sems2.at[off - 1],
                recv_sem=recv_sems2.at[my],
                device_id=(p,),
                device_id_type=pl.DeviceIdType.MESH,
            )
            rdma.start()
            phase2.append(rdma)

        for off in range(1, N_DEV):
            q = (my - off) % N_DEV
            recv = pltpu.make_async_remote_copy(
                src_ref=out_ref.at[pl.ds(q * CHUNK, CHUNK), :],
                dst_ref=out_ref.at[pl.ds(q * CHUNK, CHUNK), :],
                send_sem=send_sems2.at[0],
                recv_sem=recv_sems2.at[q],
                device_id=(q,),
                device_id_type=pl.DeviceIdType.MESH,
            )
            recv.wait_recv()
        for rdma in phase1:
            rdma.wait_send()
        for rdma in phase2:
            rdma.wait_send()

    return pl.pallas_call(
        body,
        out_shape=jax.ShapeDtypeStruct((M, N), jnp.float32),
        in_specs=[
            pl.BlockSpec(memory_space=pltpu.VMEM),
            pl.BlockSpec(memory_space=pltpu.VMEM),
        ],
        out_specs=pl.BlockSpec(memory_space=pltpu.VMEM),
        scratch_shapes=[
            pltpu.VMEM((M, N), jnp.float32),
            pltpu.VMEM((N_DEV, CHUNK, N), jnp.float32),
            pltpu.SemaphoreType.DMA((N_DEV,)),
            pltpu.SemaphoreType.DMA((N_DEV,)),
            pltpu.SemaphoreType.DMA((N_DEV,)),
            pltpu.SemaphoreType.DMA((N_DEV,)),
        ],
        compiler_params=pltpu.CompilerParams(collective_id=0),
    )(A, B)


# device time: 48859 ns/iter; 1.3027x vs baseline; 1.2262x over previous
import jax
import jax.numpy as jnp
from jax import lax
from jax.experimental import pallas as pl
from jax.experimental.pallas import tpu as pltpu

N_DEV = 16
PLANE = 4
ZDIM = 4
M = 768
N = 768
CHUNK = M // N_DEV
RBLOCK = M // PLANE

N_SEND = 21


def kernel(A, B):
    def body(a_ref, b_ref, out_ref, acc_ref, prbuf, pbuf, zbuf,
             send_sems, prsems, zrsems, csems):
        my = lax.axis_index("i")
        z = my // PLANE
        p = my % PLANE

        def plane_peer(off):
            return z * PLANE + (p + off) % PLANE

        def col_peer(off):
            return ((z + off) % ZDIM) * PLANE + p

        barrier_sem = pltpu.get_barrier_semaphore()
        for off in range(1, PLANE):
            pl.semaphore_signal(
                barrier_sem, inc=1, device_id=(plane_peer(off),),
                device_id_type=pl.DeviceIdType.MESH,
            )
        for off in range(1, ZDIM):
            pl.semaphore_signal(
                barrier_sem, inc=1, device_id=(col_peer(off),),
                device_id_type=pl.DeviceIdType.MESH,
            )
        pl.semaphore_wait(barrier_sem, 6)

        acc_ref[...] = jnp.dot(
            a_ref[...], b_ref[...], preferred_element_type=jnp.float32
        )

        sends = []
        slot = 0

        for off in range(1, PLANE):
            pp = (p + off) % PLANE
            rdma = pltpu.make_async_remote_copy(
                src_ref=acc_ref.at[pl.ds(pp * RBLOCK, RBLOCK), :],
                dst_ref=pbuf.at[p],
                send_sem=send_sems.at[slot],
                recv_sem=prsems.at[p],
                device_id=(plane_peer(off),),
                device_id_type=pl.DeviceIdType.MESH,
            )
            rdma.start()
            sends.append(rdma)
            slot += 1

        total = acc_ref[pl.ds(p * RBLOCK, RBLOCK), :]
        for off in range(1, PLANE):
            pp = (p - off) % PLANE
            recv = pltpu.make_async_remote_copy(
                src_ref=pbuf.at[pp], dst_ref=pbuf.at[pp],
                send_sem=send_sems.at[0], recv_sem=prsems.at[pp],
                device_id=(my,), device_id_type=pl.DeviceIdType.MESH,
            )
            recv.wait_recv()
            total = total + pbuf[pp]
        prbuf[...] = total

        for off in range(1, ZDIM):
            zz = (z + off) % ZDIM
            rdma = pltpu.make_async_remote_copy(
                src_ref=prbuf.at[pl.ds(zz * CHUNK, CHUNK), :],
                dst_ref=zbuf.at[z],
                send_sem=send_sems.at[slot],
                recv_sem=zrsems.at[z],
                device_id=(col_peer(off),),
                device_id_type=pl.DeviceIdType.MESH,
            )
            rdma.start()
            sends.append(rdma)
            slot += 1

        red = prbuf[pl.ds(z * CHUNK, CHUNK), :]
        for off in range(1, ZDIM):
            zz = (z - off) % ZDIM
            recv = pltpu.make_async_remote_copy(
                src_ref=zbuf.at[zz], dst_ref=zbuf.at[zz],
                send_sem=send_sems.at[0], recv_sem=zrsems.at[zz],
                device_id=(my,), device_id_type=pl.DeviceIdType.MESH,
            )
            recv.wait_recv()
            red = red + zbuf[zz]
        c = p * PLANE + z
        out_ref[pl.ds(c * CHUNK, CHUNK), :] = red

        for off in range(1, ZDIM):
            rdma = pltpu.make_async_remote_copy(
                src_ref=out_ref.at[pl.ds(c * CHUNK, CHUNK), :],
                dst_ref=out_ref.at[pl.ds(c * CHUNK, CHUNK), :],
                send_sem=send_sems.at[slot],
                recv_sem=csems.at[c],
                device_id=(col_peer(off),),
                device_id_type=pl.DeviceIdType.MESH,
            )
            rdma.start()
            sends.append(rdma)
            slot += 1
        for off in range(1, PLANE):
            rdma = pltpu.make_async_remote_copy(
                src_ref=out_ref.at[pl.ds(c * CHUNK, CHUNK), :],
                dst_ref=out_ref.at[pl.ds(c * CHUNK, CHUNK), :],
                send_sem=send_sems.at[slot],
                recv_sem=csems.at[c],
                device_id=(plane_peer(off),),
                device_id_type=pl.DeviceIdType.MESH,
            )
            rdma.start()
            sends.append(rdma)
            slot += 1

        for off in range(1, ZDIM):
            zz = (z - off) % ZDIM
            cc = p * PLANE + zz
            recv = pltpu.make_async_remote_copy(
                src_ref=out_ref.at[pl.ds(cc * CHUNK, CHUNK), :],
                dst_ref=out_ref.at[pl.ds(cc * CHUNK, CHUNK), :],
                send_sem=send_sems.at[0], recv_sem=csems.at[cc],
                device_id=(my,), device_id_type=pl.DeviceIdType.MESH,
            )
            recv.wait_recv()
            for poff in range(1, PLANE):
                rdma = pltpu.make_async_remote_copy(
                    src_ref=out_ref.at[pl.ds(cc * CHUNK, CHUNK), :],
                    dst_ref=out_ref.at[pl.ds(cc * CHUNK, CHUNK), :],
                    send_sem=send_sems.at[slot],
                    recv_sem=csems.at[cc],
                    device_id=(plane_peer(poff),),
                    device_id_type=pl.DeviceIdType.MESH,
                )
                rdma.start()
                sends.append(rdma)
                slot += 1

        for off in range(1, PLANE):
            pp = (p - off) % PLANE
            for zz in range(ZDIM):
                cc = pp * PLANE + zz
                recv = pltpu.make_async_remote_copy(
                    src_ref=out_ref.at[pl.ds(cc * CHUNK, CHUNK), :],
                    dst_ref=out_ref.at[pl.ds(cc * CHUNK, CHUNK), :],
                    send_sem=send_sems.at[0], recv_sem=csems.at[cc],
                    device_id=(my,), device_id_type=pl.DeviceIdType.MESH,
                )
                recv.wait_recv()

        for rdma in sends:
            rdma.wait_send()

    return pl.pallas_call(
        body,
        out_shape=jax.ShapeDtypeStruct((M, N), jnp.float32),
        in_specs=[
            pl.BlockSpec(memory_space=pltpu.VMEM),
            pl.BlockSpec(memory_space=pltpu.VMEM),
        ],
        out_specs=pl.BlockSpec(memory_space=pltpu.VMEM),
        scratch_shapes=[
            pltpu.VMEM((M, N), jnp.float32),
            pltpu.VMEM((RBLOCK, N), jnp.float32),
            pltpu.VMEM((PLANE, RBLOCK, N), jnp.float32),
            pltpu.VMEM((ZDIM, CHUNK, N), jnp.float32),
            pltpu.SemaphoreType.DMA((N_SEND,)),
            pltpu.SemaphoreType.DMA((PLANE,)),
            pltpu.SemaphoreType.DMA((ZDIM,)),
            pltpu.SemaphoreType.DMA((N_DEV,)),
        ],
        compiler_params=pltpu.CompilerParams(collective_id=0),
    )(A, B)


# device time: 42058 ns/iter; 1.5133x vs baseline; 1.1617x over previous
import jax
import jax.numpy as jnp
from jax import lax
from jax.experimental import pallas as pl
from jax.experimental.pallas import tpu as pltpu

N_DEV = 16
PLANE = 4
ZDIM = 4
M = 768
N = 768
CHUNK = M // N_DEV
RBLOCK = M // PLANE

N_SEND = 30


def kernel(A, B):
    def body(a_ref, b_ref, out_ref, acc_ref, prbuf, pbuf, zbuf,
             send_sems, prsems, zrsems, csems):
        my = lax.axis_index("i")
        z = my // PLANE
        p = my % PLANE

        def plane_peer(off):
            return z * PLANE + (p + off) % PLANE

        def col_peer(off):
            return ((z + off) % ZDIM) * PLANE + p

        barrier_sem = pltpu.get_barrier_semaphore()
        for off in range(1, PLANE):
            pl.semaphore_signal(
                barrier_sem, inc=1, device_id=(plane_peer(off),),
                device_id_type=pl.DeviceIdType.MESH,
            )
        for off in range(1, ZDIM):
            pl.semaphore_signal(
                barrier_sem, inc=1, device_id=(col_peer(off),),
                device_id_type=pl.DeviceIdType.MESH,
            )
        pl.semaphore_wait(barrier_sem, 6)

        acc_ref[...] = jnp.dot(
            a_ref[...], b_ref[...], preferred_element_type=jnp.float32
        )

        sends = []
        slot = 0

        zz_order = [1, 2, 3, 0]
        for zoff in zz_order:
            zz = (z + zoff) % ZDIM
            for off in range(1, PLANE):
                pp = (p + off) % PLANE
                rdma = pltpu.make_async_remote_copy(
                    src_ref=acc_ref.at[
                        pl.ds(pp * RBLOCK + zz * CHUNK, CHUNK), :],
                    dst_ref=pbuf.at[p, pl.ds(zz * CHUNK, CHUNK), :],
                    send_sem=send_sems.at[slot],
                    recv_sem=prsems.at[p * ZDIM + zz],
                    device_id=(plane_peer(off),),
                    device_id_type=pl.DeviceIdType.MESH,
                )
                rdma.start()
                sends.append(rdma)
                slot += 1

        for zoff in zz_order:
            zz = (z + zoff) % ZDIM
            sub = acc_ref[pl.ds(p * RBLOCK + zz * CHUNK, CHUNK), :]
            for off in range(1, PLANE):
                pp = (p - off) % PLANE
                recv = pltpu.make_async_remote_copy(
                    src_ref=pbuf.at[pp, pl.ds(zz * CHUNK, CHUNK), :],
                    dst_ref=pbuf.at[pp, pl.ds(zz * CHUNK, CHUNK), :],
                    send_sem=send_sems.at[0],
                    recv_sem=prsems.at[pp * ZDIM + zz],
                    device_id=(my,), device_id_type=pl.DeviceIdType.MESH,
                )
                recv.wait_recv()
                sub = sub + pbuf[pp, pl.ds(zz * CHUNK, CHUNK), :]
            prbuf[pl.ds(zz * CHUNK, CHUNK), :] = sub
            if zoff != 0:
                rdma = pltpu.make_async_remote_copy(
                    src_ref=prbuf.at[pl.ds(zz * CHUNK, CHUNK), :],
                    dst_ref=zbuf.at[z],
                    send_sem=send_sems.at[slot],
                    recv_sem=zrsems.at[z],
                    device_id=(col_peer(zoff),),
                    device_id_type=pl.DeviceIdType.MESH,
                )
                rdma.start()
                sends.append(rdma)
                slot += 1

        red = prbuf[pl.ds(z * CHUNK, CHUNK), :]
        for off in range(1, ZDIM):
            zz = (z - off) % ZDIM
            recv = pltpu.make_async_remote_copy(
                src_ref=zbuf.at[zz], dst_ref=zbuf.at[zz],
                send_sem=send_sems.at[0], recv_sem=zrsems.at[zz],
                device_id=(my,), device_id_type=pl.DeviceIdType.MESH,
            )
            recv.wait_recv()
            red = red + zbuf[zz]
        c = p * PLANE + z
        out_ref[pl.ds(c * CHUNK, CHUNK), :] = red

        for off in range(1, ZDIM):
            rdma = pltpu.make_async_remote_copy(
                src_ref=out_ref.at[pl.ds(c * CHUNK, CHUNK), :],
                dst_ref=out_ref.at[pl.ds(c * CHUNK, CHUNK), :],
                send_sem=send_sems.at[slot],
                recv_sem=csems.at[c],
                device_id=(col_peer(off),),
                device_id_type=pl.DeviceIdType.MESH,
            )
            rdma.start()
            sends.append(rdma)
            slot += 1
        for off in range(1, PLANE):
            rdma = pltpu.make_async_remote_copy(
                src_ref=out_ref.at[pl.ds(c * CHUNK, CHUNK), :],
                dst_ref=out_ref.at[pl.ds(c * CHUNK, CHUNK), :],
                send_sem=send_sems.at[slot],
                recv_sem=csems.at[c],
                device_id=(plane_peer(off),),
                device_id_type=pl.DeviceIdType.MESH,
            )
            rdma.start()
            sends.append(rdma)
            slot += 1

        for off in range(1, ZDIM):
            zz = (z - off) % ZDIM
            cc = p * PLANE + zz
            recv = pltpu.make_async_remote_copy(
                src_ref=out_ref.at[pl.ds(cc * CHUNK, CHUNK), :],
                dst_ref=out_ref.at[pl.ds(cc * CHUNK, CHUNK), :],
                send_sem=send_sems.at[0], recv_sem=csems.at[cc],
                device_id=(my,), device_id_type=pl.DeviceIdType.MESH,
            )
            recv.wait_recv()
            for poff in range(1, PLANE):
                rdma = pltpu.make_async_remote_copy(
                    src_ref=out_ref.at[pl.ds(cc * CHUNK, CHUNK), :],
                    dst_ref=out_ref.at[pl.ds(cc * CHUNK, CHUNK), :],
                    send_sem=send_sems.at[slot],
                    recv_sem=csems.at[cc],
                    device_id=(plane_peer(poff),),
                    device_id_type=pl.DeviceIdType.MESH,
                )
                rdma.start()
                sends.append(rdma)
                slot += 1

        for off in range(1, PLANE):
            pp = (p - off) % PLANE
            for zz in range(ZDIM):
                cc = pp * PLANE + zz
                recv = pltpu.make_async_remote_copy(
                    src_ref=out_ref.at[pl.ds(cc * CHUNK, CHUNK), :],
                    dst_ref=out_ref.at[pl.ds(cc * CHUNK, CHUNK), :],
                    send_sem=send_sems.at[0], recv_sem=csems.at[cc],
                    device_id=(my,), device_id_type=pl.DeviceIdType.MESH,
                )
                recv.wait_recv()

        for rdma in sends:
            rdma.wait_send()

    return pl.pallas_call(
        body,
        out_shape=jax.ShapeDtypeStruct((M, N), jnp.float32),
        in_specs=[
            pl.BlockSpec(memory_space=pltpu.VMEM),
            pl.BlockSpec(memory_space=pltpu.VMEM),
        ],
        out_specs=pl.BlockSpec(memory_space=pltpu.VMEM),
        scratch_shapes=[
            pltpu.VMEM((M, N), jnp.float32),
            pltpu.VMEM((RBLOCK, N), jnp.float32),
            pltpu.VMEM((PLANE, RBLOCK, N), jnp.float32),
            pltpu.VMEM((ZDIM, CHUNK, N), jnp.float32),
            pltpu.SemaphoreType.DMA((N_SEND,)),
            pltpu.SemaphoreType.DMA((PLANE * ZDIM,)),
            pltpu.SemaphoreType.DMA((ZDIM,)),
            pltpu.SemaphoreType.DMA((N_DEV,)),
        ],
        compiler_params=pltpu.CompilerParams(collective_id=0),
    )(A, B)
